# baseline (speedup 1.0000x reference)
import jax
import jax.numpy as jnp
from jax import lax
from jax.experimental import pallas as pl
from jax.experimental.pallas import tpu as pltpu
from jax.experimental.pallas import tpu_sc as plsc

D = 64
NC, NS = 2, 16
NW = NC * NS
CHUNK = 128
SCALE = 1.0 / (D ** 0.5)


def _make_kernel(n_rows):
    per_w = n_rows // NW
    n_chunks = per_w // CHUNK
    mesh = plsc.VectorSubcoreMesh(core_axis_name="c", subcore_axis_name="s")

    def body(idx_hbm, table_hbm, out_hbm, idx_v, rows_v, gsem, ssem):
        wid = lax.axis_index("s") * NC + lax.axis_index("c")
        pltpu.sync_copy(idx_hbm.at[wid], idx_v)

        def chunk_body(g, _):
            pltpu.async_copy(table_hbm.at[idx_v.at[g]], rows_v, gsem).wait()

            def scale_row(i, _):
                for j in range(D // 16):
                    sl = pl.ds(j * 16, 16)
                    rows_v[i, sl] = rows_v[i, sl] * SCALE
                return 0

            lax.fori_loop(0, CHUNK, scale_row, 0)
            pltpu.async_copy(rows_v, out_hbm.at[wid, g], ssem).wait()
            return 0

        lax.fori_loop(0, n_chunks, chunk_body, 0)

    return pl.kernel(
        body,
        out_type=jax.ShapeDtypeStruct((NW, n_chunks, CHUNK, D), jnp.float32),
        mesh=mesh,
        compiler_params=pltpu.CompilerParams(use_tc_tiling_on_sc=False),
        scratch_types=[
            pltpu.VMEM((n_chunks, CHUNK), jnp.int32),
            pltpu.VMEM((CHUNK, D), jnp.float32),
            pltpu.SemaphoreType.DMA,
            pltpu.SemaphoreType.DMA,
        ],
    )


def kernel(x, table):
    B, S = x.shape
    n = B * S
    idx = x.reshape(NW, (n // NW) // CHUNK, CHUNK).astype(jnp.int32)
    out = _make_kernel(n)(idx, table)
    return out.reshape(B, S, D)


# speedup vs baseline: 1.1896x; 1.1896x over previous
import jax
import jax.numpy as jnp
from jax import lax
from jax.experimental import pallas as pl
from jax.experimental.pallas import tpu as pltpu
from jax.experimental.pallas import tpu_sc as plsc

D = 64
NC, NS = 2, 16
NW = NC * NS
CHUNK = 128
SCALE = 1.0 / (D ** 0.5)


def _make_kernel(n_rows):
    per_w = n_rows // NW
    n_chunks = per_w // CHUNK
    assert n_chunks % 2 == 0
    mesh = plsc.VectorSubcoreMesh(core_axis_name="c", subcore_axis_name="s")

    def body(idx_hbm, table_hbm, out_hbm, idx_v,
             gbuf0, gbuf1, sbuf0, sbuf1, gsem0, gsem1, ssem0, ssem1):
        wid = lax.axis_index("s") * NC + lax.axis_index("c")
        pltpu.sync_copy(idx_hbm.at[wid], idx_v)

        gbufs, sbufs = (gbuf0, gbuf1), (sbuf0, sbuf1)
        gsems, ssems = (gsem0, gsem1), (ssem0, ssem1)

        def start_gather(g, b):
            pltpu.async_copy(table_hbm.at[idx_v.at[g]], gbufs[b], gsems[b])

        def wait_gather(b):
            pltpu.make_async_copy(
                table_hbm.at[idx_v.at[0]], gbufs[b], gsems[b]).wait()

        def start_scatter(g, b):
            pltpu.async_copy(sbufs[b], out_hbm.at[wid, g], ssems[b])

        def wait_scatter(b):
            pltpu.make_async_copy(
                sbufs[b], out_hbm.at[wid, 0], ssems[b]).wait()

        start_gather(0, 0)
        start_gather(1, 1)

        def step(g, b):
            wait_gather(b)
            gbuf, sbuf = gbufs[b], sbufs[b]

            @plsc.parallel_loop(0, CHUNK, unroll=4)
            def _(i):
                for j in range(D // 16):
                    sl = pl.ds(j * 16, 16)
                    sbuf[i, sl] = gbuf[i, sl] * SCALE

            @pl.when(g + 2 < n_chunks)
            def _():
                start_gather(g + 2, b)

            @pl.when(g >= 2)
            def _():
                wait_scatter(b)

            start_scatter(g, b)

        def pair(t, _):
            step(2 * t, 0)
            step(2 * t + 1, 1)
            return 0

        lax.fori_loop(0, n_chunks // 2, pair, 0)
        wait_scatter(0)
        wait_scatter(1)

    return pl.kernel(
        body,
        out_type=jax.ShapeDtypeStruct((NW, n_chunks, CHUNK, D), jnp.float32),
        mesh=mesh,
        compiler_params=pltpu.CompilerParams(use_tc_tiling_on_sc=False),
        scratch_types=[
            pltpu.VMEM((n_chunks, CHUNK), jnp.int32),
            pltpu.VMEM((CHUNK, D), jnp.float32),
            pltpu.VMEM((CHUNK, D), jnp.float32),
            pltpu.VMEM((CHUNK, D), jnp.float32),
            pltpu.VMEM((CHUNK, D), jnp.float32),
            pltpu.SemaphoreType.DMA,
            pltpu.SemaphoreType.DMA,
            pltpu.SemaphoreType.DMA,
            pltpu.SemaphoreType.DMA,
        ],
    )


def kernel(x, table):
    B, S = x.shape
    n = B * S
    idx = x.reshape(NW, (n // NW) // CHUNK, CHUNK).astype(jnp.int32)
    out = _make_kernel(n)(idx, table)
    return out.reshape(B, S, D)


# speedup vs baseline: 1.2054x; 1.0132x over previous
import jax
import jax.numpy as jnp
from jax import lax
from jax.experimental import pallas as pl
from jax.experimental.pallas import tpu as pltpu
from jax.experimental.pallas import tpu_sc as plsc

D = 64
NC, NS = 2, 16
NW = NC * NS
SCALE = 1.0 / (D ** 0.5)
SPLIT = (104, 96)


def _make_kernel(B, S):
    rows_per_w = B // NW
    assert rows_per_w % 2 == 0 and sum(SPLIT) == S
    mesh = plsc.VectorSubcoreMesh(core_axis_name="c", subcore_axis_name="s")

    def body(x_hbm, table_hbm, out_hbm, idx_v,
             gbuf0, gbuf1, sbuf0, sbuf1, gsem0, gsem1, ssem0, ssem1):
        wid = lax.axis_index("s") * NC + lax.axis_index("c")
        row0 = wid * rows_per_w
        pltpu.sync_copy(x_hbm.at[pl.ds(row0, rows_per_w)], idx_v)

        gbufs, sbufs = (gbuf0, gbuf1), (sbuf0, sbuf1)
        gsems, ssems = (gsem0, gsem1), (ssem0, ssem1)

        def gather_descs(r, b):
            descs = []
            s0 = 0
            for w in SPLIT:
                descs.append(pltpu.make_async_copy(
                    table_hbm.at[idx_v.at[r, pl.ds(s0, w)]],
                    gbufs[b].at[pl.ds(s0, w)], gsems[b]))
                s0 += w
            return descs

        def start_gather(r, b):
            for d in gather_descs(r, b):
                d.start()

        def wait_gather(b):
            for d in gather_descs(0, b):
                d.wait()

        def start_scatter(r, b):
            pltpu.async_copy(sbufs[b], out_hbm.at[row0 + r], ssems[b])

        def wait_scatter(b):
            pltpu.make_async_copy(sbufs[b], out_hbm.at[row0], ssems[b]).wait()

        start_gather(0, 0)
        start_gather(1, 1)

        def step(r, b):
            wait_gather(b)

            @pl.when(r >= 2)
            def _():
                wait_scatter(b)

            gbuf, sbuf = gbufs[b], sbufs[b]

            @plsc.parallel_loop(0, S, unroll=4)
            def _(i):
                for j in range(D // 16):
                    sl = pl.ds(j * 16, 16)
                    sbuf[i, sl] = gbuf[i, sl] * SCALE

            @pl.when(r + 2 < rows_per_w)
            def _():
                start_gather(r + 2, b)

            start_scatter(r, b)

        def pair(t, _):
            step(2 * t, 0)
            step(2 * t + 1, 1)
            return 0

        lax.fori_loop(0, rows_per_w // 2, pair, 0)
        wait_scatter(0)
        wait_scatter(1)

    return pl.kernel(
        body,
        out_type=jax.ShapeDtypeStruct((B, S, D), jnp.float32),
        mesh=mesh,
        compiler_params=pltpu.CompilerParams(use_tc_tiling_on_sc=False),
        scratch_types=[
            pltpu.VMEM((rows_per_w, S), jnp.int32),
            pltpu.VMEM((S, D), jnp.float32),
            pltpu.VMEM((S, D), jnp.float32),
            pltpu.VMEM((S, D), jnp.float32),
            pltpu.VMEM((S, D), jnp.float32),
            pltpu.SemaphoreType.DMA,
            pltpu.SemaphoreType.DMA,
            pltpu.SemaphoreType.DMA,
            pltpu.SemaphoreType.DMA,
        ],
    )


def kernel(x, table):
    B, S = x.shape
    return _make_kernel(B, S)(x.astype(jnp.int32), table)
